# baseline (speedup 1.0000x reference)
import functools

import jax
import jax.numpy as jnp
from jax import lax
from jax.experimental import pallas as pl
from jax.experimental.pallas import tpu as pltpu
from jax.experimental.pallas import tpu_sc as plsc

B = 16384
NC, NS, L = 2, 16, 16
NW = NC * NS
ROWS = B // NW
OUT_W = ROWS * 4

_mesh = plsc.VectorSubcoreMesh(core_axis_name="c", subcore_axis_name="s")


@functools.partial(
    pl.kernel,
    out_type=jax.ShapeDtypeStruct((B * 4,), jnp.float32),
    mesh=_mesh,
    scratch_types=[
        pltpu.VMEM((ROWS,), jnp.float32),
        pltpu.VMEM((OUT_W,), jnp.float32),
    ],
    compiler_params=pltpu.CompilerParams(needs_layout_passes=False),
)
def _build_tril(vec_hbm, out_hbm, v_vmem, o_vmem):
    wid = lax.axis_index("s") * NC + lax.axis_index("c")
    pltpu.sync_copy(vec_hbm.at[pl.ds(wid * ROWS, ROWS)], v_vmem)
    lane = lax.iota(jnp.int32, L)
    m = lane % 4
    const = jnp.where((m == 0) | (m == 3), 1.0, 0.0).astype(jnp.float32)
    for g in range(ROWS // L):
        base = g * 4 * L
        for k in range(4):
            o_vmem[pl.ds(base + k * L, L)] = const
        v = v_vmem[pl.ds(g * L, L)]
        plsc.store_scatter(o_vmem, [lane * 4 + (base + 2)], v)
    pltpu.sync_copy(o_vmem, out_hbm.at[pl.ds(wid * OUT_W, OUT_W)])


def kernel(vec):
    flat = _build_tril(vec.reshape(-1))
    return flat.reshape(B, 2, 2)


# speedup vs baseline: 1.0067x; 1.0067x over previous
import functools

import jax
import jax.numpy as jnp
from jax import lax
from jax.experimental import pallas as pl
from jax.experimental.pallas import tpu as pltpu
from jax.experimental.pallas import tpu_sc as plsc

B = 16384
NC, NS, L = 2, 16, 16
NW = NC * NS
ROWS = B // NW
OUT_W = ROWS * 4

_mesh = plsc.VectorSubcoreMesh(core_axis_name="c", subcore_axis_name="s")


@functools.partial(
    pl.kernel,
    out_type=jax.ShapeDtypeStruct((B * 4,), jnp.float32),
    mesh=_mesh,
    scratch_types=[
        pltpu.VMEM((ROWS,), jnp.float32),
        pltpu.VMEM((OUT_W,), jnp.float32),
    ],
    compiler_params=pltpu.CompilerParams(
        needs_layout_passes=False, skip_device_barrier=True
    ),
)
def _build_tril(vec_hbm, out_hbm, v_vmem, o_vmem):
    wid = lax.axis_index("s") * NC + lax.axis_index("c")
    pltpu.sync_copy(vec_hbm.at[pl.ds(wid * ROWS, ROWS)], v_vmem)
    lane = lax.iota(jnp.int32, L)
    m = lane % 4
    const = jnp.where((m == 0) | (m == 3), 1.0, 0.0).astype(jnp.float32)
    for g in range(ROWS // L):
        base = g * 4 * L
        for k in range(4):
            o_vmem[pl.ds(base + k * L, L)] = const
        v = v_vmem[pl.ds(g * L, L)]
        plsc.store_scatter(o_vmem, [lane * 4 + (base + 2)], v)
    pltpu.sync_copy(o_vmem, out_hbm.at[pl.ds(wid * OUT_W, OUT_W)])


def kernel(vec):
    flat = _build_tril(vec.reshape(-1))
    return flat.reshape(B, 2, 2)


# speedup vs baseline: 1.0384x; 1.0315x over previous
import functools

import jax
import jax.numpy as jnp
from jax import lax
from jax.experimental import pallas as pl
from jax.experimental.pallas import tpu as pltpu
from jax.experimental.pallas import tpu_sc as plsc

B = 16384
NS, L = 16, 16
ROWS = B // NS
OUT_W = ROWS * 4

_mesh = plsc.VectorSubcoreMesh(
    core_axis_name="c", subcore_axis_name="s", num_cores=1
)


@functools.partial(
    pl.kernel,
    out_type=jax.ShapeDtypeStruct((B * 4,), jnp.float32),
    mesh=_mesh,
    scratch_types=[
        pltpu.VMEM((ROWS,), jnp.float32),
        pltpu.VMEM((OUT_W,), jnp.float32),
    ],
    compiler_params=pltpu.CompilerParams(
        needs_layout_passes=False, skip_device_barrier=True
    ),
)
def _build_tril(vec_hbm, out_hbm, v_vmem, o_vmem):
    sid = lax.axis_index("s")
    pltpu.sync_copy(vec_hbm.at[pl.ds(sid * ROWS, ROWS)], v_vmem)
    lane = lax.iota(jnp.int32, L)
    m = lane % 4
    const = jnp.where((m == 0) | (m == 3), 1.0, 0.0).astype(jnp.float32)
    for g in range(ROWS // L):
        base = g * 4 * L
        for k in range(4):
            o_vmem[pl.ds(base + k * L, L)] = const
        v = v_vmem[pl.ds(g * L, L)]
        plsc.store_scatter(o_vmem, [lane * 4 + (base + 2)], v)
    pltpu.sync_copy(o_vmem, out_hbm.at[pl.ds(sid * OUT_W, OUT_W)])


def kernel(vec):
    flat = _build_tril(vec.reshape(-1))
    return flat.reshape(B, 2, 2)
